# Initial kernel scaffold; baseline (speedup 1.0000x reference)
import jax
import jax.numpy as jnp
from jax.experimental import pallas as pl


def kernel(user_bought_items, items, query_words, word_table, item_table, W_q, b_q, W_attn, b_attn, w_reduce):
    raise NotImplementedError("write your pallas kernel here")


# speedup vs baseline: 1.4152x; 1.4152x over previous
import functools

import jax
import jax.numpy as jnp
from jax import lax
from jax.experimental import pallas as pl
from jax.experimental.pallas import tpu as pltpu
from jax.experimental.pallas import tpu_sc as plsc

E = 32
H = 16
HIST = 50
QW = 20

NC = 2
NS = 16
NW = NC * NS
CH = 128


def _make_sc_gather(n_rows: int, nbuf: int = 4):
    assert n_rows % (NW * CH) == 0
    n_ch = n_rows // (NW * CH)
    nbuf = min(nbuf, n_ch)
    assert n_ch % nbuf == 0
    per_w = n_rows // NW
    mesh = plsc.VectorSubcoreMesh(core_axis_name="c", subcore_axis_name="s")

    @functools.partial(
        pl.kernel,
        mesh=mesh,
        out_type=jax.ShapeDtypeStruct((n_rows, E), jnp.float32),
        scratch_types=[
            pltpu.VMEM((nbuf, CH), jnp.int32),
            pltpu.VMEM((nbuf, CH, E), jnp.float32),
            pltpu.SemaphoreType.DMA,
            pltpu.SemaphoreType.DMA,
        ],
        compiler_params=pltpu.CompilerParams(use_tc_tiling_on_sc=False),
    )
    def gather_k(table_hbm, idx_hbm, out_hbm, idx_v, rows_v, gsem, wsem):
        wid = lax.axis_index("s") * NC + lax.axis_index("c")
        base = wid * per_w

        def block(bi, carry):
            row0 = base + bi * (nbuf * CH)
            gathers = []
            for b in range(nbuf):
                pltpu.sync_copy(idx_hbm.at[pl.ds(row0 + b * CH, CH)], idx_v.at[b])
                gathers.append(
                    pltpu.async_copy(table_hbm.at[idx_v.at[b]], rows_v.at[b], gsem)
                )
            writes = []
            for b in range(nbuf):
                gathers[b].wait()
                writes.append(
                    pltpu.async_copy(
                        rows_v.at[b], out_hbm.at[pl.ds(row0 + b * CH, CH)], wsem
                    )
                )
            for w in writes:
                w.wait()
            return carry

        lax.fori_loop(0, n_ch // nbuf, block, 0)

    return gather_k


def _tc_combiner(ub3, w3, wq_t, bq2, wa_t, ba2, r_mat):
    b_total = ub3.shape[0]
    bb = 256
    grid = (b_total // bb,)

    def body(ub_ref, w_ref, wq_ref, bq_ref, wa_ref, ba_ref, r_ref, out_ref):
        ub = ub_ref[...]
        qmean = jnp.mean(w_ref[...], axis=1)
        q = jnp.tanh(
            jnp.dot(qmean, wq_ref[...], preferred_element_type=jnp.float32)
            + bq_ref[...]
        )
        pq = jnp.tanh(
            jnp.dot(q, wa_ref[...], preferred_element_type=jnp.float32)
            + ba_ref[...]
        )
        pqw = jnp.dot(pq, r_ref[...], preferred_element_type=jnp.float32)
        scores = jnp.sum(ub * pqw[:, None, :], axis=2)
        m = jnp.max(scores, axis=1, keepdims=True)
        e = jnp.exp(scores - m)
        w = e / jnp.sum(e, axis=1, keepdims=True)
        user = jnp.sum(w[:, :, None] * ub, axis=1)
        out_ref[...] = q + user

    return pl.pallas_call(
        body,
        grid=grid,
        in_specs=[
            pl.BlockSpec((bb, HIST, E), lambda i: (i, 0, 0)),
            pl.BlockSpec((bb, QW, E), lambda i: (i, 0, 0)),
            pl.BlockSpec((E, E), lambda i: (0, 0)),
            pl.BlockSpec((1, E), lambda i: (0, 0)),
            pl.BlockSpec((E, E * H), lambda i: (0, 0)),
            pl.BlockSpec((1, E * H), lambda i: (0, 0)),
            pl.BlockSpec((E * H, E), lambda i: (0, 0)),
        ],
        out_specs=pl.BlockSpec((bb, E), lambda i: (i, 0)),
        out_shape=jax.ShapeDtypeStruct((b_total, E), jnp.float32),
    )(ub3, w3, wq_t, bq2, wa_t, ba2, r_mat)


def kernel(user_bought_items, items, query_words, word_table, item_table,
           W_q, b_q, W_attn, b_attn, w_reduce):
    b_total = items.shape[0]
    ub_idx = user_bought_items.reshape(-1).astype(jnp.int32)
    qw_idx = query_words.reshape(-1).astype(jnp.int32)
    it_idx = items.astype(jnp.int32)

    ub_rows = _make_sc_gather(b_total * HIST)(item_table, ub_idx)
    w_rows = _make_sc_gather(b_total * QW)(word_table, qw_idx)
    item_emb = _make_sc_gather(b_total)(item_table, it_idx)

    r_mat = jnp.kron(jnp.eye(E, dtype=jnp.float32), w_reduce[0][:, None])
    pm = _tc_combiner(
        ub_rows.reshape(b_total, HIST, E),
        w_rows.reshape(b_total, QW, E),
        W_q.T,
        b_q[None, :],
        W_attn.T,
        b_attn[None, :],
        r_mat,
    )
    return (pm, item_emb)
